# Initial kernel scaffold; baseline (speedup 1.0000x reference)
import jax
import jax.numpy as jnp
from jax.experimental import pallas as pl


def kernel(x, edge_index, batch, W1, a_src1, a_dst1, b1, W2, a_src2, a_dst2, b2, lin1_w, lin1_b, lin2_w, lin2_b):
    raise NotImplementedError("write your pallas kernel here")


# speedup vs baseline: 29.2834x; 29.2834x over previous
import functools

import jax
import jax.numpy as jnp
from jax import lax
from jax.experimental import pallas as pl
from jax.experimental.pallas import tpu as pltpu
from jax.experimental.pallas import tpu_sc as plsc

NC = 2
NS = 16
NW = NC * NS
L = 16

CHUNK = 128

_GDN = lax.GatherDimensionNumbers(
    offset_dims=(), collapsed_slice_dims=(0,), start_index_map=(0,))


def _vreg_gather(vec, idx):
    return lax.gather(vec, idx[:, None], dimension_numbers=_GDN,
                      slice_sizes=(1,),
                      mode=lax.GatherScatterMode.PROMISE_IN_BOUNDS)


def _tc_layer1(x_ref, w_ref, asrc_ref, adst_ref, h_ref, aux_ref):
    h = jnp.dot(x_ref[...], w_ref[...], preferred_element_type=jnp.float32)
    h_ref[...] = h
    a = jnp.sum(h * asrc_ref[...][None, :], axis=1)
    b = jnp.sum(h * adst_ref[...][None, :], axis=1)
    aux_ref[...] = jnp.concatenate([a, b], axis=0)


def _tc_layer2(n_nodes, num_ref, den_ref, b1_ref, w2_ref, asrc_ref,
               adst_ref, h_ref, aux_ref):
    num = (num_ref[0] + num_ref[1])[:n_nodes]
    den = (den_ref[0] + den_ref[1])[:n_nodes]
    o = num / (den + 1e-16)[:, None] + b1_ref[...][None, :]
    o = jnp.maximum(o, 0.0)
    h = jnp.dot(o, w2_ref[...], preferred_element_type=jnp.float32)
    h_ref[...] = h
    a = jnp.sum(h * asrc_ref[...][None, :], axis=1)
    b = jnp.sum(h * adst_ref[...][None, :], axis=1)
    aux_ref[...] = jnp.concatenate([a, b], axis=0)


def _tc_final(n_nodes, num_ref, den_ref, b2_ref, l1w_ref, l1b_ref, l2w_ref,
              l2b_ref, out_ref):
    num = (num_ref[0] + num_ref[1])[:n_nodes]
    den = (den_ref[0] + den_ref[1])[:n_nodes]
    o = num / (den + 1e-16)[:, None] + b2_ref[...][None, :]
    o = jnp.maximum(o, 0.0)
    t = jnp.dot(o, l1w_ref[...], preferred_element_type=jnp.float32)
    t = jnp.maximum(t + l1b_ref[...][None, :], 0.0)
    logits = jnp.dot(t, l2w_ref[...], preferred_element_type=jnp.float32)
    logits = logits + l2b_ref[...][None, :]
    m = jnp.max(logits, axis=1, keepdims=True)
    s = logits - m
    lse = jnp.log(jnp.sum(jnp.exp(s), axis=1, keepdims=True))
    out_ref[...] = s - lse


def _sc_gat_body(n_nodes, np_pad, n_chunks,
                 src_hbm, dst_hbm, aux_hbm, h_hbm,
                 num_hbm, den_hbm,
                 src_v, dst_v, aux_v, hbuf, exbuf, num_acc, den_acc, sem):
    core = lax.axis_index("c")
    sub = lax.axis_index("s")
    wid = sub * NC + core

    stripe = np_pad // NS

    if True:
        def zrow(e, _):
            for r in range(8):
                hbuf[e, pl.ds(r * L, L)] = jnp.zeros((L,), jnp.float32)
            return 0
        lax.fori_loop(0, CHUNK, zrow, 0)

        n_full = stripe // CHUNK
        rem = stripe - n_full * CHUNK

        def zcopy(i, _):
            pltpu.sync_copy(hbuf, num_acc.at[pl.ds(sub * stripe + i * CHUNK,
                                                   CHUNK)])
            return 0
        lax.fori_loop(0, n_full, zcopy, 0)
        if rem:
            pltpu.sync_copy(
                hbuf.at[pl.ds(0, rem)],
                num_acc.at[pl.ds(sub * stripe + n_full * CHUNK, rem)])

        def zex(j, _):
            exbuf[pl.ds(j * L, L)] = jnp.zeros((L,), jnp.float32)
            return 0
        lax.fori_loop(0, CHUNK // L, zex, 0)

        nd_chunks = np_pad // CHUNK
        for k in range((nd_chunks + NS - 1) // NS):
            i = sub + k * NS

            @pl.when(i < nd_chunks)
            def _():
                pltpu.sync_copy(exbuf, den_acc.at[pl.ds(i * CHUNK, CHUNK)])

        pltpu.sync_copy(aux_hbm, aux_v)
        plsc.subcore_barrier()

        base_chunks = n_chunks // NW
        extra = n_chunks - base_chunks * NW
        my_chunks = base_chunks + jnp.where(wid < extra, 1, 0)


        def chunk_body(i, _):
            c = i * NW + wid
            base = c * CHUNK
            pltpu.sync_copy(src_hbm.at[pl.ds(base, CHUNK)], src_v)
            pltpu.sync_copy(dst_hbm.at[pl.ds(base, CHUNK)], dst_v)
            pltpu.async_copy(h_hbm.at[src_v], hbuf, sem).wait()

            def grp(j, _):
                s16 = src_v[pl.ds(j * L, L)]
                d16 = dst_v[pl.ds(j * L, L)]
                a1 = plsc.load_gather(aux_v, [s16])
                a2 = plsc.load_gather(aux_v, [d16 + n_nodes])
                e = a1 + a2
                e = jnp.where(e >= 0.0, e, 0.2 * e)
                ex = jnp.exp(e)
                exbuf[pl.ds(j * L, L)] = ex
                for i16 in range(L):
                    bex = _vreg_gather(ex, jnp.full((L,), i16, jnp.int32))
                    erow = j * L + i16
                    for r in range(8):
                        hbuf[erow, pl.ds(r * L, L)] = (
                            hbuf[erow, pl.ds(r * L, L)] * bex)
                return 0
            lax.fori_loop(0, CHUNK // L, grp, 0)

            pltpu.sync_copy(hbuf, num_acc.at[dst_v], add=True)
            pltpu.sync_copy(exbuf, den_acc.at[dst_v], add=True)
            return 0
        lax.fori_loop(0, my_chunks, chunk_body, 0)

        plsc.subcore_barrier()

        pltpu.sync_copy(num_acc.at[pl.ds(sub * stripe, stripe)],
                        num_hbm.at[core, pl.ds(sub * stripe, stripe)])

        @pl.when(sub == 0)
        def _():
            pltpu.sync_copy(den_acc, den_hbm.at[pl.ds(core * np_pad, np_pad)])


def _make_sc_gat(n_nodes, np_pad, n_edges):
    n_chunks = n_edges // CHUNK
    assert n_edges % CHUNK == 0
    assert np_pad % (NS * 8) == 0 and np_pad % CHUNK == 0
    mesh = plsc.VectorSubcoreMesh(core_axis_name="c", subcore_axis_name="s")
    return pl.kernel(
        functools.partial(_sc_gat_body, n_nodes, np_pad, n_chunks),
        out_type=(
            jax.ShapeDtypeStruct((NC, np_pad, 128), jnp.float32),
            jax.ShapeDtypeStruct((NC * np_pad,), jnp.float32),
        ),
        mesh=mesh,
        compiler_params=pltpu.CompilerParams(needs_layout_passes=False),
        scratch_types=(
            pltpu.VMEM((CHUNK,), jnp.int32),
            pltpu.VMEM((CHUNK,), jnp.int32),
            pltpu.VMEM((2 * n_nodes,), jnp.float32),
            pltpu.VMEM((CHUNK, 128), jnp.float32),
            pltpu.VMEM((CHUNK,), jnp.float32),
            pltpu.VMEM_SHARED((np_pad, 128), jnp.float32),
            pltpu.VMEM_SHARED((np_pad,), jnp.float32),
            pltpu.SemaphoreType.DMA,
        ),
    )


def kernel(x, edge_index, batch, W1, a_src1, a_dst1, b1, W2, a_src2, a_dst2,
           b2, lin1_w, lin1_b, lin2_w, lin2_b):
    n_nodes, d_feat = x.shape
    n_edges = edge_index.shape[1]
    hidden = W1.shape[1]
    n_classes = lin2_w.shape[1]

    src = edge_index[0].astype(jnp.int32)
    dst = edge_index[1].astype(jnp.int32)

    np_pad = ((n_nodes + NS * 8 - 1) // (NS * 8)) * (NS * 8)
    assert np_pad % CHUNK == 0

    tc1 = pl.pallas_call(
        _tc_layer1,
        out_shape=(
            jax.ShapeDtypeStruct((n_nodes, hidden), jnp.float32),
            jax.ShapeDtypeStruct((2 * n_nodes,), jnp.float32),
        ),
    )
    h1, aux1 = tc1(x, W1, a_src1, a_dst1)

    sc_gat = _make_sc_gat(n_nodes, np_pad, n_edges)
    num1, den1 = sc_gat(src, dst, aux1, h1)
    den1 = den1.reshape(NC, np_pad)

    tc2 = pl.pallas_call(
        functools.partial(_tc_layer2, n_nodes),
        out_shape=(
            jax.ShapeDtypeStruct((n_nodes, hidden), jnp.float32),
            jax.ShapeDtypeStruct((2 * n_nodes,), jnp.float32),
        ),
    )
    h2, aux2 = tc2(num1, den1, b1, W2, a_src2, a_dst2)

    num2, den2 = sc_gat(src, dst, aux2, h2)
    den2 = den2.reshape(NC, np_pad)

    tc3 = pl.pallas_call(
        functools.partial(_tc_final, n_nodes),
        out_shape=jax.ShapeDtypeStruct((n_nodes, n_classes), jnp.float32),
    )
    return tc3(num2, den2, b2, lin1_w, lin1_b, lin2_w, lin2_b)
